# baseline (speedup 1.0000x reference)
import functools

import jax
import jax.numpy as jnp
from jax.experimental import pallas as pl
from jax.experimental.pallas import tpu as pltpu

_BM = 288


def _gather_dot_loop(idx_ref, xt_ref, xb_ref, w_ref, acc_ref, g_dst, g_src,
                     local_off, k, nb, fin):
    bm = g_dst.shape[1]

    def body(ib, carry):
        base = ib * 8
        g_dst[0, pl.ds(base, 8), :] = xb_ref[pl.ds(local_off + base, 8), :]
        for j in range(1, k):
            rows = [xt_ref[pl.ds(idx_ref[local_off + base + r, j], 1), :]
                    for r in range(8)]
            g_dst[j, pl.ds(base, 8), :] = jnp.concatenate(rows, axis=0)
        n = ib // k
        j2 = ib % k
        part = jnp.dot(g_src[j2, :, pl.ds(n * fin, fin)],
                       w_ref[pl.ds(j2 * fin, fin), :],
                       preferred_element_type=jnp.float32)
        acc_ref[pl.ds(n * bm, bm), :] += part
        return carry

    jax.lax.fori_loop(0, bm // 8, body, 0)


def _flush(acc_ref, b_ref, out_ref, block, nb):
    bm = acc_ref.shape[0] // nb
    off = block * bm
    for n in range(nb):
        acc = acc_ref[pl.ds(n * bm, bm), :] + b_ref[...]
        out_ref[n, pl.ds(off, bm), :] = jnp.where(acc > 0, acc, jnp.exp(acc) - 1.0)
    acc_ref[...] = jnp.zeros_like(acc_ref)


def _fused_body(idx_ref, xt_ref, xb_ref, w_ref, b_ref, out_ref,
                ga_ref, gb_ref, acc_ref):
    k = idx_ref.shape[1]
    nb = out_ref.shape[0]
    fin = w_ref.shape[0] // k
    s = pl.program_id(0)
    ns = pl.num_programs(0)
    nblk = 2 * (ns - 1)

    _gather_dot_loop(idx_ref, xt_ref, xb_ref, w_ref, acc_ref, ga_ref, gb_ref,
                     0, k, nb, fin)
    _flush(acc_ref, b_ref, out_ref,
           jnp.minimum(jnp.maximum(2 * s - 1, 0), nblk - 1), nb)

    _gather_dot_loop(idx_ref, xt_ref, xb_ref, w_ref, acc_ref, gb_ref, ga_ref,
                     _BM, k, nb, fin)

    @pl.when(s < ns - 1)
    def _():
        _flush(acc_ref, b_ref, out_ref, 2 * s, nb)


@jax.jit
def kernel(x, index_list, W, b):
    n, m, fin = x.shape
    kf, fout = W.shape
    k = index_list.shape[1]
    nf = n * fin
    nblk = (m + _BM - 1) // _BM
    nblk = nblk + (nblk % 2)
    mp = nblk * _BM

    xt = jnp.pad(x.transpose(1, 0, 2).reshape(m, nf), ((0, mp - m), (0, 0)))
    idxp = jnp.pad(index_list, ((0, mp - m), (0, 0)), constant_values=m)
    b2 = b.reshape(1, fout)

    out = pl.pallas_call(
        _fused_body,
        grid=(nblk // 2 + 1,),
        in_specs=[
            pl.BlockSpec((2 * _BM, k),
                         lambda s, _h=nblk // 2: (jnp.minimum(s, _h - 1), 0),
                         memory_space=pltpu.SMEM),
            pl.BlockSpec((mp, nf), lambda s: (0, 0)),
            pl.BlockSpec((2 * _BM, nf),
                         lambda s, _h=nblk // 2: (jnp.minimum(s, _h - 1), 0)),
            pl.BlockSpec((kf, fout), lambda s: (0, 0)),
            pl.BlockSpec((1, fout), lambda s: (0, 0)),
        ],
        out_specs=pl.BlockSpec((n, mp, fout), lambda s: (0, 0, 0)),
        out_shape=jax.ShapeDtypeStruct((n, mp, fout), jnp.float32),
        scratch_shapes=[
            pltpu.VMEM((k, _BM, nf), jnp.float32),
            pltpu.VMEM((k, _BM, nf), jnp.float32),
            pltpu.VMEM((n * _BM, fout), jnp.float32),
        ],
        compiler_params=pltpu.CompilerParams(
            dimension_semantics=("arbitrary",)),
    )(idxp, xt, xt, W, b2)
    return out[:, :m]


# speedup vs baseline: 1.3596x; 1.3596x over previous
import functools

import jax
import jax.numpy as jnp
from jax import lax
from jax.experimental import pallas as pl
from jax.experimental.pallas import tpu as pltpu
from jax.experimental.pallas import tpu_sc as plsc

_CHUNK = 128


def _sc_gather(table, idxf, nw, nc):
    rows, nf = idxf.shape[0], table.shape[1]
    b_per_w = rows // nw
    nchunks = b_per_w // _CHUNK
    mesh = plsc.VectorSubcoreMesh(core_axis_name="c", subcore_axis_name="s")

    @functools.partial(
        pl.kernel, mesh=mesh,
        out_type=jax.ShapeDtypeStruct((rows, nf), jnp.float32),
        scratch_types=[
            pltpu.VMEM((_CHUNK,), jnp.int32),
            pltpu.VMEM((_CHUNK, nf), jnp.float32),
            pltpu.SemaphoreType.DMA,
        ],
    )
    def gather(table_hbm, idx_hbm, out_hbm, idx_v, rows_v, sem):
        wid = lax.axis_index("s") * nc + lax.axis_index("c")

        def chunk(i, carry):
            base = wid * b_per_w + i * _CHUNK
            pltpu.sync_copy(idx_hbm.at[pl.ds(base, _CHUNK)], idx_v)
            pltpu.async_copy(table_hbm.at[idx_v], rows_v, sem).wait()
            pltpu.sync_copy(rows_v, out_hbm.at[pl.ds(base, _CHUNK)])
            return carry

        jax.lax.fori_loop(0, nchunks, chunk, 0)

    return gather(table, idxf)


def _tc_body(g_ref, xb_ref, w_ref, b_ref, out_ref):
    kk = g_ref.shape[0] + 1
    nb, bm, fout = out_ref.shape
    fin = w_ref.shape[0] // kk

    for n in range(nb):
        acc = jnp.dot(xb_ref[:, n * fin:(n + 1) * fin], w_ref[0:fin, :],
                      preferred_element_type=jnp.float32)
        for j in range(1, kk):
            acc = acc + jnp.dot(g_ref[j - 1, :, n * fin:(n + 1) * fin],
                                w_ref[j * fin:(j + 1) * fin, :],
                                preferred_element_type=jnp.float32)
        acc = acc + b_ref[...]
        out_ref[n] = jnp.where(acc > 0, acc, jnp.exp(acc) - 1.0)


@jax.jit
def kernel(x, index_list, W, b):
    n, m, fin = x.shape
    kf, fout = W.shape
    k = index_list.shape[1]
    nf = n * fin
    bm = 512

    info = plsc.get_sparse_core_info()
    nc, ns = info.num_cores, info.num_subcores
    nw = nc * ns

    mp = m
    while (mp % bm) or (((k - 1) * mp) % (nw * _CHUNK)):
        mp += 16
    xt = jnp.pad(x.transpose(1, 0, 2).reshape(m, nf), ((0, mp - m), (0, 0)))
    idxp = jnp.pad(index_list, ((0, mp - m), (0, 0)), constant_values=m)
    idxf = idxp[:, 1:].T.reshape(-1)
    b2 = b.reshape(1, fout)

    g = _sc_gather(xt, idxf, nw, nc).reshape(k - 1, mp, nf)

    out = pl.pallas_call(
        _tc_body,
        grid=(mp // bm,),
        in_specs=[
            pl.BlockSpec((k - 1, bm, nf), lambda j: (0, j, 0)),
            pl.BlockSpec((bm, nf), lambda j: (j, 0)),
            pl.BlockSpec((kf, fout), lambda j: (0, 0)),
            pl.BlockSpec((1, fout), lambda j: (0, 0)),
        ],
        out_specs=pl.BlockSpec((n, bm, fout), lambda j: (0, j, 0)),
        out_shape=jax.ShapeDtypeStruct((n, mp, fout), jnp.float32),
        compiler_params=pltpu.CompilerParams(
            dimension_semantics=("arbitrary",)),
    )(g, xt, W, b2)
    return out[:, :m]


# speedup vs baseline: 1.7948x; 1.3202x over previous
import functools

import jax
import jax.numpy as jnp
from jax import lax
from jax.experimental import pallas as pl
from jax.experimental.pallas import tpu as pltpu
from jax.experimental.pallas import tpu_sc as plsc

_CHUNK = 128
_BM = 512
_SPLIT = 10


def _sc_gather(table, idxf, nw, nc):
    rows, nf = idxf.shape[0], table.shape[1]
    b_per_w = rows // nw
    nchunks = b_per_w // _CHUNK
    mesh = plsc.VectorSubcoreMesh(core_axis_name="c", subcore_axis_name="s")

    @functools.partial(
        pl.kernel, mesh=mesh,
        out_type=jax.ShapeDtypeStruct((rows, nf), jnp.float32),
        scratch_types=[
            pltpu.VMEM((_CHUNK,), jnp.int32),
            pltpu.VMEM((_CHUNK, nf), jnp.float32),
            pltpu.SemaphoreType.DMA,
        ],
    )
    def gather(table_hbm, idx_hbm, out_hbm, idx_v, rows_v, sem):
        wid = lax.axis_index("s") * nc + lax.axis_index("c")

        def chunk(i, carry):
            base = wid * b_per_w + i * _CHUNK
            pltpu.sync_copy(idx_hbm.at[pl.ds(base, _CHUNK)], idx_v)
            pltpu.async_copy(table_hbm.at[idx_v], rows_v, sem).wait()
            pltpu.sync_copy(rows_v, out_hbm.at[pl.ds(base, _CHUNK)])
            return carry

        jax.lax.fori_loop(0, nchunks, chunk, 0)

    return gather(table, idxf)


def _dots(g_at, xb_ref, w_ref, b_ref, out_ref, k, nb, fin):
    for n in range(nb):
        acc = jnp.dot(xb_ref[:, n * fin:(n + 1) * fin], w_ref[0:fin, :],
                      preferred_element_type=jnp.float32)
        for j in range(1, k):
            acc = acc + jnp.dot(g_at(j)[:, n * fin:(n + 1) * fin],
                                w_ref[j * fin:(j + 1) * fin, :],
                                preferred_element_type=jnp.float32)
        acc = acc + b_ref[...]
        out_ref[n] = jnp.where(acc > 0, acc, jnp.exp(acc) - 1.0)


def _fused_body(idx_ref, xt_ref, xb_ref, w_ref, b_ref, out_ref, g_ref):
    k = idx_ref.shape[1]
    nb, bm, fout = out_ref.shape
    fin = w_ref.shape[0] // k

    def gather_group(ib, carry):
        base = ib * 8
        for j in range(1, k):
            rows = [xt_ref[pl.ds(idx_ref[base + r, j], 1), :] for r in range(8)]
            g_ref[j - 1, pl.ds(base, 8), :] = jnp.concatenate(rows, axis=0)
        return carry

    jax.lax.fori_loop(0, bm // 8, gather_group, 0, unroll=2)
    _dots(lambda j: g_ref[j - 1], xb_ref, w_ref, b_ref, out_ref, k, nb, fin)


def _gemm_body(g_ref, xb_ref, w_ref, b_ref, out_ref):
    k = g_ref.shape[0] + 1
    nb, bm, fout = out_ref.shape
    fin = w_ref.shape[0] // k
    _dots(lambda j: g_ref[j - 1], xb_ref, w_ref, b_ref, out_ref, k, nb, fin)


@jax.jit
def kernel(x, index_list, W, b):
    n, m, fin = x.shape
    kf, fout = W.shape
    k = index_list.shape[1]
    nf = n * fin

    info = plsc.get_sparse_core_info()
    nc, ns = info.num_cores, info.num_subcores
    nw = nc * ns

    mp = ((m + 1 + _BM - 1) // _BM) * _BM
    m1 = _SPLIT * _BM
    m2 = mp - m1
    xt = jnp.pad(x.transpose(1, 0, 2).reshape(m, nf), ((0, mp - m), (0, 0)))
    idxp = jnp.pad(index_list, ((0, mp - m), (0, 0)), constant_values=m)
    idxf2 = idxp[m1:, 1:].T.reshape(-1)
    b2 = b.reshape(1, fout)

    g2 = _sc_gather(xt, idxf2, nw, nc).reshape(k - 1, m2, nf)

    out1 = pl.pallas_call(
        _fused_body,
        grid=(m1 // _BM,),
        in_specs=[
            pl.BlockSpec((_BM, k), lambda j: (j, 0), memory_space=pltpu.SMEM),
            pl.BlockSpec((mp, nf), lambda j: (0, 0)),
            pl.BlockSpec((_BM, nf), lambda j: (j, 0)),
            pl.BlockSpec((kf, fout), lambda j: (0, 0)),
            pl.BlockSpec((1, fout), lambda j: (0, 0)),
        ],
        out_specs=pl.BlockSpec((n, _BM, fout), lambda j: (0, j, 0)),
        out_shape=jax.ShapeDtypeStruct((n, m1, fout), jnp.float32),
        scratch_shapes=[pltpu.VMEM((k - 1, _BM, nf), jnp.float32)],
        compiler_params=pltpu.CompilerParams(
            dimension_semantics=("arbitrary",)),
    )(idxp[:m1], xt, xt, W, b2)

    out2 = pl.pallas_call(
        _gemm_body,
        grid=(m2 // _BM,),
        in_specs=[
            pl.BlockSpec((k - 1, _BM, nf), lambda j: (0, j, 0)),
            pl.BlockSpec((_BM, nf), lambda j, _o=_SPLIT: (j + _o, 0)),
            pl.BlockSpec((kf, fout), lambda j: (0, 0)),
            pl.BlockSpec((1, fout), lambda j: (0, 0)),
        ],
        out_specs=pl.BlockSpec((n, _BM, fout), lambda j: (0, j, 0)),
        out_shape=jax.ShapeDtypeStruct((n, m2, fout), jnp.float32),
        compiler_params=pltpu.CompilerParams(
            dimension_semantics=("arbitrary",)),
    )(g2, xt, W, b2)

    return jnp.concatenate([out1, out2], axis=1)[:, :m]


# speedup vs baseline: 1.9362x; 1.0788x over previous
import functools

import jax
import jax.numpy as jnp
from jax import lax
from jax.experimental import pallas as pl
from jax.experimental.pallas import tpu as pltpu
from jax.experimental.pallas import tpu_sc as plsc

_CHUNK = 128
_BM = 512
_SPLIT = 12


def _sc_gather(table, idxf, nw, nc):
    rows, nf = idxf.shape[0], table.shape[1]
    b_per_w = rows // nw
    nchunks = b_per_w // _CHUNK
    mesh = plsc.VectorSubcoreMesh(core_axis_name="c", subcore_axis_name="s")

    @functools.partial(
        pl.kernel, mesh=mesh,
        out_type=jax.ShapeDtypeStruct((rows, nf), jnp.float32),
        scratch_types=[
            pltpu.VMEM((_CHUNK,), jnp.int32),
            pltpu.VMEM((_CHUNK, nf), jnp.float32),
            pltpu.SemaphoreType.DMA,
        ],
    )
    def gather(table_hbm, idx_hbm, out_hbm, idx_v, rows_v, sem):
        wid = lax.axis_index("s") * nc + lax.axis_index("c")

        def chunk(i, carry):
            base = wid * b_per_w + i * _CHUNK
            pltpu.sync_copy(idx_hbm.at[pl.ds(base, _CHUNK)], idx_v)
            pltpu.async_copy(table_hbm.at[idx_v], rows_v, sem).wait()
            pltpu.sync_copy(rows_v, out_hbm.at[pl.ds(base, _CHUNK)])
            return carry

        jax.lax.fori_loop(0, nchunks, chunk, 0)

    return gather(table, idxf)


def _dots(g_at, xb_ref, w_ref, b_ref, out_ref, k, nb, fin):
    for n in range(nb):
        acc = jnp.dot(xb_ref[:, n * fin:(n + 1) * fin], w_ref[0:fin, :],
                      preferred_element_type=jnp.float32)
        for j in range(1, k):
            acc = acc + jnp.dot(g_at(j)[:, n * fin:(n + 1) * fin],
                                w_ref[j * fin:(j + 1) * fin, :],
                                preferred_element_type=jnp.float32)
        acc = acc + b_ref[...]
        out_ref[n] = jnp.where(acc > 0, acc, jnp.exp(acc) - 1.0)


def _fused_body(idx_ref, xt_ref, xb_ref, w_ref, b_ref, out_ref, g_ref):
    k = idx_ref.shape[1]
    nb, bm, fout = out_ref.shape
    fin = w_ref.shape[0] // k

    def gather_group(ib, carry):
        base = ib * 8
        for j in range(1, k):
            rows = [xt_ref[pl.ds(idx_ref[base + r, j], 1), :] for r in range(8)]
            g_ref[j - 1, pl.ds(base, 8), :] = jnp.concatenate(rows, axis=0)
        return carry

    jax.lax.fori_loop(0, bm // 8, gather_group, 0, unroll=2)
    _dots(lambda j: g_ref[j - 1], xb_ref, w_ref, b_ref, out_ref, k, nb, fin)


def _gemm_body(g_ref, xb_ref, w_ref, b_ref, out_ref):
    k = g_ref.shape[0] + 1
    nb, bm, fout = out_ref.shape
    fin = w_ref.shape[0] // k
    _dots(lambda j: g_ref[j - 1], xb_ref, w_ref, b_ref, out_ref, k, nb, fin)


@jax.jit
def kernel(x, index_list, W, b):
    n, m, fin = x.shape
    kf, fout = W.shape
    k = index_list.shape[1]
    nf = n * fin

    info = plsc.get_sparse_core_info()
    nc, ns = info.num_cores, info.num_subcores
    nw = nc * ns

    mp = ((m + 1 + _BM - 1) // _BM) * _BM
    m1 = _SPLIT * _BM
    m2 = mp - m1
    xt = jnp.pad(x.transpose(1, 0, 2).reshape(m, nf), ((0, mp - m), (0, 0)))
    idxp = jnp.pad(index_list, ((0, mp - m), (0, 0)), constant_values=m)
    idxf2 = idxp[m1:, 1:].T.reshape(-1)
    b2 = b.reshape(1, fout)

    g2 = _sc_gather(xt, idxf2, nw, nc).reshape(k - 1, m2, nf)

    out1 = pl.pallas_call(
        _fused_body,
        grid=(m1 // _BM,),
        in_specs=[
            pl.BlockSpec((_BM, k), lambda j: (j, 0), memory_space=pltpu.SMEM),
            pl.BlockSpec((mp, nf), lambda j: (0, 0)),
            pl.BlockSpec((_BM, nf), lambda j: (j, 0)),
            pl.BlockSpec((kf, fout), lambda j: (0, 0)),
            pl.BlockSpec((1, fout), lambda j: (0, 0)),
        ],
        out_specs=pl.BlockSpec((n, _BM, fout), lambda j: (0, j, 0)),
        out_shape=jax.ShapeDtypeStruct((n, m1, fout), jnp.float32),
        scratch_shapes=[pltpu.VMEM((k - 1, _BM, nf), jnp.float32)],
        compiler_params=pltpu.CompilerParams(
            dimension_semantics=("arbitrary",)),
    )(idxp[:m1], xt, xt, W, b2)

    out2 = pl.pallas_call(
        _gemm_body,
        grid=(m2 // _BM,),
        in_specs=[
            pl.BlockSpec((k - 1, _BM, nf), lambda j: (0, j, 0)),
            pl.BlockSpec((_BM, nf), lambda j, _o=_SPLIT: (j + _o, 0)),
            pl.BlockSpec((kf, fout), lambda j: (0, 0)),
            pl.BlockSpec((1, fout), lambda j: (0, 0)),
        ],
        out_specs=pl.BlockSpec((n, _BM, fout), lambda j: (0, j, 0)),
        out_shape=jax.ShapeDtypeStruct((n, m2, fout), jnp.float32),
        compiler_params=pltpu.CompilerParams(
            dimension_semantics=("arbitrary",)),
    )(g2, xt, W, b2)

    return jnp.concatenate([out1, out2], axis=1)[:, :m]
